# baseline (speedup 1.0000x reference)
import jax
import jax.numpy as jnp
from jax import lax
from jax.experimental import pallas as pl
from jax.experimental.pallas import tpu as pltpu
from jax.experimental.pallas import tpu_sc as plsc

_DIM = 64
_BATCH = 16384
_NEG = 20
_KP = 32
_PW = 128
_OW = 24

_NC = 2
_NS = 16
_NW = _NC * _NS
_BT = _BATCH // _NW
_CH = 16
_NCHUNK = _BT // _CH
_IW = 64
_NEG_ROWS = _CH * _NEG
_NEG_G = _NEG_ROWS // _IW
_TN = _BT * _NEG


def _sc_body(ip_ref, op_ref, on_ref, po_ref, win_ref, wout_ref, y_ref,
             ii_div, ip_div, in_div, poff,
             in_a, pos_a, neg_a, in_b, pos_b, neg_b, y_v, sem_a, sem_b):
  wid = lax.axis_index("s") * _NC + lax.axis_index("c")
  lane = lax.iota(jnp.int32, 16)
  bufs_a = (in_a, pos_a, neg_a, sem_a)
  bufs_b = (in_b, pos_b, neg_b, sem_b)

  pltpu.sync_copy(ip_ref.at[pl.ds(wid * _BT, _BT)], ii_div)
  pltpu.sync_copy(op_ref.at[pl.ds(wid * _BT, _BT)], ip_div)
  pltpu.sync_copy(on_ref.at[pl.ds(wid * _TN, _TN)], in_div)
  pltpu.sync_copy(po_ref.at[pl.ds(wid * _BT * _OW, _BT * _OW)], poff)

  def issue(c, bufs):
    in_r, pos_r, neg_r, sem = bufs
    pltpu.async_copy(win_ref.at[ii_div.at[pl.ds(c * _CH, _CH)]], in_r, sem)
    pltpu.async_copy(wout_ref.at[ip_div.at[pl.ds(c * _CH, _CH)]], pos_r, sem)
    for j in range(_NEG_G):
      pltpu.async_copy(
          wout_ref.at[in_div.at[pl.ds(c * _NEG_ROWS + j * _IW, _IW)]],
          neg_r.at[pl.ds(j * _IW, _IW)], sem)

  def drain(bufs):
    in_r, pos_r, neg_r, sem = bufs
    pltpu.make_async_copy(win_ref.at[ii_div.at[pl.ds(0, _CH)]], in_r, sem).wait()
    pltpu.make_async_copy(wout_ref.at[ip_div.at[pl.ds(0, _CH)]], pos_r, sem).wait()
    for j in range(_NEG_G):
      pltpu.make_async_copy(wout_ref.at[in_div.at[pl.ds(0, _IW)]],
                            neg_r.at[pl.ds(j * _IW, _IW)], sem).wait()

  def compute(c, bufs):
    in_r, pos_r, neg_r, _ = bufs

    @plsc.parallel_loop(0, _CH, unroll=2)
    def elem_body(e):
      ge = c * _CH + e
      v0 = poff[pl.ds(ge * _OW, 16)]
      v1 = poff[pl.ds(ge * _OW + 8, 16)]
      iv = [in_r[e, pl.ds(v0[0] + 16 * j, 16)] for j in range(4)]

      def dot(rows_ref, r, off):
        acc = iv[0] * rows_ref[r, pl.ds(off, 16)]
        for j in range(1, 4):
          acc += iv[j] * rows_ref[r, pl.ds(off + 16 * j, 16)]
        return jnp.sum(acc)

      def noff(k):
        return v0[k + 2] if k <= 13 else v1[k - 6]

      vec_a = jnp.full((16,), dot(pos_r, e, v0[1]), jnp.float32)
      for k in range(15):
        s = -dot(neg_r, e * _NEG + k, noff(k))
        vec_a = jnp.where(lane == k + 1, jnp.full((16,), s, jnp.float32), vec_a)
      vec_b = jnp.zeros((16,), jnp.float32)
      for k in range(15, _NEG):
        s = -dot(neg_r, e * _NEG + k, noff(k))
        vec_b = jnp.where(lane == k - 15, jnp.full((16,), s, jnp.float32), vec_b)

      y_v[pl.ds(e * _KP, 16)] = vec_a
      y_v[pl.ds(e * _KP + 16, 16)] = vec_b

    pltpu.sync_copy(y_v, y_ref.at[pl.ds((wid * _BT + c * _CH) * _KP, _CH * _KP)])

  issue(0, bufs_a)

  def pair_body(g, carry):
    c0 = 2 * g
    issue(c0 + 1, bufs_b)
    drain(bufs_a)
    compute(c0, bufs_a)
    issue(jnp.minimum(c0 + 2, _NCHUNK - 1), bufs_a)
    drain(bufs_b)
    compute(c0 + 1, bufs_b)
    return carry

  lax.fori_loop(0, _NCHUNK // 2, pair_body, 0)
  drain(bufs_a)


def _loss_body(y_ref, o_ref):
  x = jnp.clip(y_ref[...], -10.0, 10.0)
  ls = jnp.minimum(x, 0.0) - jnp.log1p(jnp.exp(-jnp.abs(x)))
  col = lax.broadcasted_iota(jnp.int32, y_ref.shape, 1) % _KP
  ls = jnp.where(col < _NEG + 1, ls, 0.0)
  o_ref[0, 0] = -jnp.sum(ls) * (1.0 / _BATCH)


@jax.jit
def kernel(input_pos, output_pos, output_neg, W_in, W_out):
  ii = jax.lax.shift_right_logical(input_pos, 1)
  ip = jax.lax.shift_right_logical(output_pos, 1)
  on = jax.lax.shift_right_logical(output_neg.reshape(_BATCH * _NEG), 1)
  cat = jnp.concatenate(
      [input_pos[:, None], output_pos[:, None], output_neg], axis=1)
  po = jnp.pad((cat & 1) * _DIM, ((0, 0), (0, _OW - _NEG - 2)))
  po = po.reshape(_BATCH * _OW)
  w_in2 = W_in.reshape(-1, _PW)
  w_out2 = W_out.reshape(-1, _PW)

  mesh = plsc.VectorSubcoreMesh(core_axis_name="c", subcore_axis_name="s")
  y = pl.kernel(
      _sc_body,
      out_type=jax.ShapeDtypeStruct((_BATCH * _KP,), jnp.float32),
      mesh=mesh,
      compiler_params=pltpu.CompilerParams(needs_layout_passes=False),
      scratch_types=[
          pltpu.VMEM((_BT,), jnp.int32),
          pltpu.VMEM((_BT,), jnp.int32),
          pltpu.VMEM((_TN,), jnp.int32),
          pltpu.VMEM((_BT * _OW,), jnp.int32),
          pltpu.VMEM((_CH, _PW), jnp.float32),
          pltpu.VMEM((_CH, _PW), jnp.float32),
          pltpu.VMEM((_NEG_ROWS, _PW), jnp.float32),
          pltpu.VMEM((_CH, _PW), jnp.float32),
          pltpu.VMEM((_CH, _PW), jnp.float32),
          pltpu.VMEM((_NEG_ROWS, _PW), jnp.float32),
          pltpu.VMEM((_CH * _KP,), jnp.float32),
          pltpu.SemaphoreType.DMA,
          pltpu.SemaphoreType.DMA,
      ],
  )(ii, ip, on, po, w_in2, w_out2)

  loss = pl.pallas_call(
      _loss_body,
      out_shape=jax.ShapeDtypeStruct((1, 1), jnp.float32),
      out_specs=pl.BlockSpec(memory_space=pltpu.SMEM),
  )(y.reshape(_BATCH * _KP // 128, 128))
  return loss[0, 0]
